# Initial kernel scaffold; baseline (speedup 1.0000x reference)
import jax
import jax.numpy as jnp
from jax.experimental import pallas as pl


def kernel(x, view_ids, visit_ids, router_view, router_visit, W1, b1, W2, b2):
    raise NotImplementedError("write your pallas kernel here")


# speedup vs baseline: 2.4774x; 2.4774x over previous
import jax
import jax.numpy as jnp
from jax import lax
from jax.experimental import pallas as pl
from jax.experimental.pallas import tpu as pltpu
from jax.experimental.pallas import tpu_sc as plsc

B, L, D = 4, 2048, 768
E = 16
D_FF = D * 4
LANES = 16
TL = 512


def _router_body(view_ids_hbm, visit_ids_hbm, router_view_hbm, router_visit_hbm,
                 esort_hbm, perm_hbm,
                 vidx_v, sidx_v, vtab_v, stab_v, ek_v, pm_v):
    c = lax.axis_index("c")
    s = lax.axis_index("s")

    @pl.when((c == 0) & (s == 0))
    def _():
        pltpu.sync_copy(view_ids_hbm, vidx_v)
        pltpu.sync_copy(visit_ids_hbm, sidx_v)
        pltpu.sync_copy(router_view_hbm, vtab_v)
        pltpu.sync_copy(router_visit_hbm, stab_v)
        lane = lax.iota(jnp.int32, LANES)

        def _bfly(v, op):
            for s in (1, 2, 4, 8):
                v = op(v, v.at[lane ^ s].get(mode="promise_in_bounds"))
            return v

        vrows = [vtab_v[pl.ds(i * LANES, LANES)] for i in range(8)]
        srows = [stab_v[pl.ds(i * LANES, LANES)] for i in range(16)]
        top1 = jnp.zeros((LANES,), jnp.int32)
        for b in range(B):
            vrow = vidx_v[pl.ds(b * LANES, LANES)]
            srow = sidx_v[pl.ds(b * LANES, LANES)]
            lv = vrows[0]
            for i in range(1, 8):
                lv = jnp.where(vrow == i, vrows[i], lv)
            sv = srows[0]
            for i in range(1, 16):
                sv = jnp.where(srow == i, srows[i], sv)
            logits = lv + sv
            mx = _bfly(logits, jnp.maximum)
            cand = jnp.where(logits == mx, lane, jnp.int32(LANES))
            idx = _bfly(cand, jnp.minimum)
            top1 = jnp.where(lane == b, idx, top1)
        esort = jnp.zeros((LANES,), jnp.int32)
        perm = jnp.zeros((LANES,), jnp.int32)
        for b in range(B):
            kb = top1.at[jnp.where(lane < LANES, lane * 0 + b, lane)].get(
                mode="promise_in_bounds")
            before = (top1 < kb) | ((top1 == kb) & (lane < b))
            cnt = jnp.where(before & (lane < B), jnp.int32(1), jnp.int32(0))
            rank = _bfly(cnt, jnp.add)
            esort = jnp.where(lane == rank, kb, esort)
            perm = jnp.where(lane == rank, jnp.int32(b), perm)
        ek_v[...] = esort
        pm_v[...] = perm
        pltpu.sync_copy(ek_v, esort_hbm)
        pltpu.sync_copy(pm_v, perm_hbm)


def _route(view_ids, visit_ids, router_view, router_visit):
    mesh = plsc.VectorSubcoreMesh(core_axis_name="c", subcore_axis_name="s")
    return pl.kernel(
        _router_body,
        out_type=(
            jax.ShapeDtypeStruct((LANES,), jnp.int32),
            jax.ShapeDtypeStruct((LANES,), jnp.int32),
        ),
        mesh=mesh,
        scratch_types=[
            pltpu.VMEM((B * LANES,), jnp.int32),
            pltpu.VMEM((B * LANES,), jnp.int32),
            pltpu.VMEM((8 * E,), jnp.float32),
            pltpu.VMEM((16 * E,), jnp.float32),
            pltpu.VMEM((LANES,), jnp.int32),
            pltpu.VMEM((LANES,), jnp.int32),
        ],
    )(view_ids, visit_ids, router_view.reshape(-1), router_visit.reshape(-1))


def _ffn_body(es_ref, pm_ref, x_ref, w1_ref, b1_ref, w2_ref, b2_ref, o_ref):
    xb = x_ref[0]
    h = jnp.maximum(
        jnp.dot(xb, w1_ref[0], preferred_element_type=jnp.float32) + b1_ref[0],
        0.0,
    )
    o_ref[0] = jnp.dot(h, w2_ref[0], preferred_element_type=jnp.float32) + b2_ref[0]


def _ffn(esort, perm, x, W1, b1, W2, b2):
    grid_spec = pltpu.PrefetchScalarGridSpec(
        num_scalar_prefetch=2,
        grid=(B, L // TL),
        in_specs=[
            pl.BlockSpec((1, TL, D), lambda b, l, es, pm: (pm[b], l, 0)),
            pl.BlockSpec((1, D, D_FF), lambda b, l, es, pm: (es[b], 0, 0)),
            pl.BlockSpec((1, 1, D_FF), lambda b, l, es, pm: (es[b], 0, 0)),
            pl.BlockSpec((1, D_FF, D), lambda b, l, es, pm: (es[b], 0, 0)),
            pl.BlockSpec((1, 1, D), lambda b, l, es, pm: (es[b], 0, 0)),
        ],
        out_specs=pl.BlockSpec((1, TL, D), lambda b, l, es, pm: (pm[b], l, 0)),
    )
    return pl.pallas_call(
        _ffn_body,
        grid_spec=grid_spec,
        out_shape=jax.ShapeDtypeStruct((B, L, D), jnp.float32),
        compiler_params=pltpu.CompilerParams(
            dimension_semantics=("arbitrary", "arbitrary"),
            vmem_limit_bytes=100 * 1024 * 1024,
        ),
    )(esort, perm, x, W1, b1.reshape(E, 1, D_FF), W2, b2.reshape(E, 1, D))


def kernel(x, view_ids, visit_ids, router_view, router_visit, W1, b1, W2, b2):
    vi = jnp.repeat(view_ids.astype(jnp.int32), LANES)
    si = jnp.repeat(visit_ids.astype(jnp.int32), LANES)
    esort, perm = _route(vi, si, router_view, router_visit)
    return _ffn(esort, perm, x, W1, b1, W2, b2)
